# Initial kernel scaffold; baseline (speedup 1.0000x reference)
import functools

import jax
import jax.numpy as jnp
from jax import lax
from jax.experimental import pallas as pl
from jax.experimental.pallas import tpu as pltpu
from jax.experimental.pallas import tpu_sc as plsc

N = 10000
E = 160000
D_IN = 256
D_H = 256
D_OUT = 4

NC, NS, L = 2, 16, 16
CH = 128
NCHUNK = E // CH
ROWS_PER_TILE = N // NS

_mesh = plsc.VectorSubcoreMesh(
    core_axis_name="c", subcore_axis_name="s", num_cores=NC, num_subcores=NS
)


def _sc_deg(dst, ones_blk, zeros_narrow):
    @functools.partial(
        pl.kernel,
        out_type=jax.ShapeDtypeStruct((NC, N, L), jnp.float32),
        mesh=_mesh,
        scratch_types=[
            pltpu.VMEM((CH,), jnp.int32),
            pltpu.VMEM((CH, L), jnp.float32),
            pltpu.VMEM_SHARED((N, L), jnp.float32),
        ],
    )
    def k(dst_hbm, ones_hbm, zero_hbm, out_hbm, idx_d, ones_v, acc):
        c = lax.axis_index("c")
        s = lax.axis_index("s")
        rows0 = s * ROWS_PER_TILE
        pltpu.sync_copy(zero_hbm, acc.at[pl.ds(rows0, ROWS_PER_TILE)])
        pltpu.sync_copy(ones_hbm, ones_v)
        plsc.subcore_barrier()

        half = NCHUNK // NC

        @pl.loop(c * half + s, (c + 1) * half, step=NS)
        def _(ch):
            pltpu.sync_copy(dst_hbm.at[pl.ds(ch * CH, CH)], idx_d)
            pltpu.sync_copy(ones_v, acc.at[idx_d], add=True)

        plsc.subcore_barrier()
        pltpu.sync_copy(
            acc.at[pl.ds(rows0, ROWS_PER_TILE)],
            out_hbm.at[c].at[pl.ds(rows0, ROWS_PER_TILE)],
        )

    return k(dst, ones_blk, zeros_narrow)


def _sc_agg128(y_flat, src, dst, zeros_wide):
    @functools.partial(
        pl.kernel,
        out_type=jax.ShapeDtypeStruct((NC, N, 128), jnp.float32),
        mesh=_mesh,
        scratch_types=[
            pltpu.VMEM((CH,), jnp.int32),
            pltpu.VMEM((CH,), jnp.int32),
            pltpu.VMEM((CH, 128), jnp.float32),
            pltpu.VMEM_SHARED((N, 128), jnp.float32),
            pltpu.SemaphoreType.DMA,
        ],
    )
    def k(y_hbm, src_hbm, dst_hbm, zero_hbm, out_hbm, idx_s, idx_d, buf, acc, sem):
        c = lax.axis_index("c")
        s = lax.axis_index("s")
        rows0 = s * ROWS_PER_TILE
        pltpu.sync_copy(zero_hbm, acc.at[pl.ds(rows0, ROWS_PER_TILE)])
        plsc.subcore_barrier()

        @pl.loop(s, NCHUNK, step=NS)
        def _(ch):
            e0 = ch * CH
            pltpu.sync_copy(src_hbm.at[pl.ds(e0, CH)], idx_s)
            pltpu.sync_copy(dst_hbm.at[pl.ds(e0, CH)], idx_d)

            @pl.when(c == 1)
            def _():
                for j in range(CH // L):
                    sl = pl.ds(j * L, L)
                    idx_s[sl] = idx_s[sl] + N

            pltpu.async_copy(y_hbm.at[idx_s], buf, sem).wait()
            pltpu.sync_copy(buf, acc.at[idx_d], add=True)

        plsc.subcore_barrier()
        pltpu.sync_copy(
            acc.at[pl.ds(rows0, ROWS_PER_TILE)],
            out_hbm.at[c].at[pl.ds(rows0, ROWS_PER_TILE)],
        )

    return k(y_flat, src, dst, zeros_wide)


def _sc_agg16(y2, src, dst, zeros_narrow):
    @functools.partial(
        pl.kernel,
        out_type=jax.ShapeDtypeStruct((NC, N, L), jnp.float32),
        mesh=_mesh,
        scratch_types=[
            pltpu.VMEM((CH,), jnp.int32),
            pltpu.VMEM((CH,), jnp.int32),
            pltpu.VMEM((CH, L), jnp.float32),
            pltpu.VMEM_SHARED((N, L), jnp.float32),
            pltpu.SemaphoreType.DMA,
        ],
    )
    def k(y_hbm, src_hbm, dst_hbm, zero_hbm, out_hbm, idx_s, idx_d, buf, acc, sem):
        c = lax.axis_index("c")
        s = lax.axis_index("s")
        rows0 = s * ROWS_PER_TILE
        pltpu.sync_copy(zero_hbm, acc.at[pl.ds(rows0, ROWS_PER_TILE)])
        plsc.subcore_barrier()

        half = NCHUNK // NC

        @pl.loop(c * half + s, (c + 1) * half, step=NS)
        def _(ch):
            e0 = ch * CH
            pltpu.sync_copy(src_hbm.at[pl.ds(e0, CH)], idx_s)
            pltpu.sync_copy(dst_hbm.at[pl.ds(e0, CH)], idx_d)
            pltpu.async_copy(y_hbm.at[idx_s], buf, sem).wait()
            pltpu.sync_copy(buf, acc.at[idx_d], add=True)

        plsc.subcore_barrier()
        pltpu.sync_copy(
            acc.at[pl.ds(rows0, ROWS_PER_TILE)],
            out_hbm.at[c].at[pl.ds(rows0, ROWS_PER_TILE)],
        )

    return k(y2, src, dst, zeros_narrow)


_RB = 1000


def _deg_inv_sqrt(dr_ref):
    deg = dr_ref[0, :, 0:1] + dr_ref[1, :, 0:1] + 1.0
    return lax.rsqrt(deg)


def _tc_matmul(x, W):
    def body(x_ref, w_ref, o_ref):
        o_ref[...] = lax.dot_general(
            x_ref[...], w_ref[...], (((1,), (0,)), ((), ())),
            precision=lax.Precision.HIGHEST,
            preferred_element_type=jnp.float32,
        )

    return pl.pallas_call(
        body,
        grid=(N // _RB,),
        in_specs=[
            pl.BlockSpec((_RB, D_IN), lambda i: (i, 0)),
            pl.BlockSpec((D_IN, D_H), lambda i: (0, 0)),
        ],
        out_specs=pl.BlockSpec((_RB, D_H), lambda i: (i, 0)),
        out_shape=jax.ShapeDtypeStruct((N, D_H), jnp.float32),
    )(x, W)


def _tc_scale_split(xw, degraw):
    def body(xw_ref, dr_ref, y_ref):
        d = _deg_inv_sqrt(dr_ref)
        sc = xw_ref[...] * d
        y_ref[0] = sc[:, :128]
        y_ref[1] = sc[:, 128:]

    return pl.pallas_call(
        body,
        grid=(N // _RB,),
        in_specs=[
            pl.BlockSpec((_RB, D_H), lambda i: (i, 0)),
            pl.BlockSpec((NC, _RB, L), lambda i: (0, i, 0)),
        ],
        out_specs=pl.BlockSpec((NC, _RB, 128), lambda i: (0, i, 0)),
        out_shape=jax.ShapeDtypeStruct((NC, N, 128), jnp.float32),
    )(xw, degraw)


def _tc_layer2(acc1, y1, degraw, b1r, W2p):
    def body(a_ref, y_ref, dr_ref, b1_ref, w2_ref, y2_ref):
        d = _deg_inv_sqrt(dr_ref)
        h0 = jnp.maximum(d * (a_ref[0] + y_ref[0]) + b1_ref[0:1, :128], 0.0)
        h1 = jnp.maximum(d * (a_ref[1] + y_ref[1]) + b1_ref[0:1, 128:], 0.0)
        hw = lax.dot_general(
            h0, w2_ref[:128], (((1,), (0,)), ((), ())),
            precision=lax.Precision.HIGHEST, preferred_element_type=jnp.float32,
        ) + lax.dot_general(
            h1, w2_ref[128:], (((1,), (0,)), ((), ())),
            precision=lax.Precision.HIGHEST, preferred_element_type=jnp.float32,
        )
        y2_ref[...] = d * hw

    return pl.pallas_call(
        body,
        grid=(N // _RB,),
        in_specs=[
            pl.BlockSpec((NC, _RB, 128), lambda i: (0, i, 0)),
            pl.BlockSpec((NC, _RB, 128), lambda i: (0, i, 0)),
            pl.BlockSpec((NC, _RB, L), lambda i: (0, i, 0)),
            pl.BlockSpec((1, D_H), lambda i: (0, 0)),
            pl.BlockSpec((D_H, L), lambda i: (0, 0)),
        ],
        out_specs=pl.BlockSpec((_RB, L), lambda i: (i, 0)),
        out_shape=jax.ShapeDtypeStruct((N, L), jnp.float32),
    )(acc1, y1, degraw, b1r, W2p)


def _tc_out(acc2, y2, degraw, b2r):
    def body(a_ref, y_ref, dr_ref, b2_ref, o_ref):
        d = _deg_inv_sqrt(dr_ref)
        logit = d * (a_ref[0] + a_ref[1] + y_ref[...]) + b2_ref[...]
        lid = lax.broadcasted_iota(jnp.int32, (_RB, L), 1)
        real = lid < D_OUT
        masked = jnp.where(real, logit, -jnp.inf)
        m = jnp.max(masked, axis=1, keepdims=True)
        e = jnp.where(real, jnp.exp(logit - m), 0.0)
        p = e / jnp.sum(e, axis=1, keepdims=True)
        o_ref[...] = p[:, :D_OUT]

    return pl.pallas_call(
        body,
        grid=(N // _RB,),
        in_specs=[
            pl.BlockSpec((NC, _RB, L), lambda i: (0, i, 0)),
            pl.BlockSpec((_RB, L), lambda i: (i, 0)),
            pl.BlockSpec((NC, _RB, L), lambda i: (0, i, 0)),
            pl.BlockSpec((1, L), lambda i: (0, 0)),
        ],
        out_specs=pl.BlockSpec((_RB, D_OUT), lambda i: (i, 0)),
        out_shape=jax.ShapeDtypeStruct((N, D_OUT), jnp.float32),
    )(acc2, y2, degraw, b2r)


def kernel(x, edge_index, W1, b1, W2, b2):
    src = edge_index[0]
    dst = edge_index[1]

    ones_blk = jnp.ones((CH, L), jnp.float32)
    zeros_narrow = jnp.zeros((ROWS_PER_TILE, L), jnp.float32)
    zeros_wide = jnp.zeros((ROWS_PER_TILE, 128), jnp.float32)
    b1r = b1.reshape(1, D_H)
    W2p = jnp.pad(W2, ((0, 0), (0, L - D_OUT)))
    b2r = jnp.pad(b2, (0, L - D_OUT)).reshape(1, L)

    degraw = _sc_deg(dst, ones_blk, zeros_narrow)
    xw = _tc_matmul(x, W1)
    y1 = _tc_scale_split(xw, degraw)
    acc1 = _sc_agg128(y1.reshape(NC * N, 128), src, dst, zeros_wide)
    y2 = _tc_layer2(acc1, y1, degraw, b1r, W2p)
    acc2 = _sc_agg16(y2, src, dst, zeros_narrow)
    return _tc_out(acc2, y2, degraw, b2r)


# speedup vs baseline: 11.7665x; 11.7665x over previous
import functools

import jax
import jax.numpy as jnp
from jax import lax
from jax.experimental import pallas as pl
from jax.experimental.pallas import tpu as pltpu
from jax.experimental.pallas import tpu_sc as plsc

N = 10000
NP = 10240
E = 160000
D_IN = 256
D_H = 256
D_OUT = 4

NC, NS, L = 2, 16, 16
CH = 128
NCHUNK = E // CH
ROWS_PER_TILE = NP // NS

_mesh = plsc.VectorSubcoreMesh(
    core_axis_name="c", subcore_axis_name="s", num_cores=NC, num_subcores=NS
)
_sc_params = pltpu.CompilerParams(use_tc_tiling_on_sc=False)


def _sc_deg(dst, ones_blk, zeros_narrow):
    @functools.partial(
        pl.kernel,
        out_type=jax.ShapeDtypeStruct((NC, NP, L), jnp.float32),
        mesh=_mesh,
        compiler_params=_sc_params,
        scratch_types=[
            pltpu.VMEM((CH,), jnp.int32),
            pltpu.VMEM((CH, L), jnp.float32),
            pltpu.VMEM_SHARED((NP, L), jnp.float32),
        ],
    )
    def k(dst_hbm, ones_hbm, zero_hbm, out_hbm, idx_d, ones_v, acc):
        c = lax.axis_index("c")
        s = lax.axis_index("s")
        rows0 = s * ROWS_PER_TILE
        pltpu.sync_copy(zero_hbm, acc.at[pl.ds(rows0, ROWS_PER_TILE)])
        pltpu.sync_copy(ones_hbm, ones_v)
        plsc.subcore_barrier()

        half = NCHUNK // NC

        @pl.loop(c * half + s, (c + 1) * half, step=NS)
        def _(ch):
            pltpu.sync_copy(dst_hbm.at[pl.ds(ch * CH, CH)], idx_d)
            pltpu.sync_copy(ones_v, acc.at[idx_d], add=True)

        plsc.subcore_barrier()
        pltpu.sync_copy(
            acc.at[pl.ds(rows0, ROWS_PER_TILE)],
            out_hbm.at[c].at[pl.ds(rows0, ROWS_PER_TILE)],
        )

    return k(dst, ones_blk, zeros_narrow)


def _sc_agg128(y_flat, src, dst, zeros_wide):
    @functools.partial(
        pl.kernel,
        out_type=jax.ShapeDtypeStruct((NC, NP, 128), jnp.float32),
        mesh=_mesh,
        compiler_params=_sc_params,
        scratch_types=[
            pltpu.VMEM((CH,), jnp.int32),
            pltpu.VMEM((CH,), jnp.int32),
            pltpu.VMEM((CH, 128), jnp.float32),
            pltpu.VMEM_SHARED((NP, 128), jnp.float32),
            pltpu.SemaphoreType.DMA,
        ],
    )
    def k(y_hbm, src_hbm, dst_hbm, zero_hbm, out_hbm, idx_s, idx_d, buf, acc, sem):
        c = lax.axis_index("c")
        s = lax.axis_index("s")
        rows0 = s * ROWS_PER_TILE
        pltpu.sync_copy(zero_hbm, acc.at[pl.ds(rows0, ROWS_PER_TILE)])
        plsc.subcore_barrier()

        @pl.loop(s, NCHUNK, step=NS)
        def _(ch):
            e0 = ch * CH
            pltpu.sync_copy(src_hbm.at[pl.ds(e0, CH)], idx_s)
            pltpu.sync_copy(dst_hbm.at[pl.ds(e0, CH)], idx_d)

            @pl.when(c == 1)
            def _():
                for j in range(CH // L):
                    sl = pl.ds(j * L, L)
                    idx_s[sl] = idx_s[sl] + NP

            pltpu.async_copy(y_hbm.at[idx_s], buf, sem).wait()
            pltpu.sync_copy(buf, acc.at[idx_d], add=True)

        plsc.subcore_barrier()
        pltpu.sync_copy(
            acc.at[pl.ds(rows0, ROWS_PER_TILE)],
            out_hbm.at[c].at[pl.ds(rows0, ROWS_PER_TILE)],
        )

    return k(y_flat, src, dst, zeros_wide)


def _sc_agg16(y2, src, dst, zeros_narrow):
    @functools.partial(
        pl.kernel,
        out_type=jax.ShapeDtypeStruct((NC, NP, L), jnp.float32),
        mesh=_mesh,
        compiler_params=_sc_params,
        scratch_types=[
            pltpu.VMEM((CH,), jnp.int32),
            pltpu.VMEM((CH,), jnp.int32),
            pltpu.VMEM((CH, L), jnp.float32),
            pltpu.VMEM_SHARED((NP, L), jnp.float32),
            pltpu.SemaphoreType.DMA,
        ],
    )
    def k(y_hbm, src_hbm, dst_hbm, zero_hbm, out_hbm, idx_s, idx_d, buf, acc, sem):
        c = lax.axis_index("c")
        s = lax.axis_index("s")
        rows0 = s * ROWS_PER_TILE
        pltpu.sync_copy(zero_hbm, acc.at[pl.ds(rows0, ROWS_PER_TILE)])
        plsc.subcore_barrier()

        half = NCHUNK // NC

        @pl.loop(c * half + s, (c + 1) * half, step=NS)
        def _(ch):
            e0 = ch * CH
            pltpu.sync_copy(src_hbm.at[pl.ds(e0, CH)], idx_s)
            pltpu.sync_copy(dst_hbm.at[pl.ds(e0, CH)], idx_d)
            pltpu.async_copy(y_hbm.at[idx_s], buf, sem).wait()
            pltpu.sync_copy(buf, acc.at[idx_d], add=True)

        plsc.subcore_barrier()
        pltpu.sync_copy(
            acc.at[pl.ds(rows0, ROWS_PER_TILE)],
            out_hbm.at[c].at[pl.ds(rows0, ROWS_PER_TILE)],
        )

    return k(y2, src, dst, zeros_narrow)


_RB = 1024


def _deg_inv_sqrt(dr_ref):
    deg = dr_ref[0, :, 0:1] + dr_ref[1, :, 0:1] + 1.0
    return lax.rsqrt(deg)


def _tc_matmul(x, W):
    def body(x_ref, w_ref, o_ref):
        o_ref[...] = lax.dot_general(
            x_ref[...], w_ref[...], (((1,), (0,)), ((), ())),
            precision=lax.Precision.HIGHEST,
            preferred_element_type=jnp.float32,
        )

    return pl.pallas_call(
        body,
        grid=(NP // _RB,),
        in_specs=[
            pl.BlockSpec((_RB, D_IN), lambda i: (i, 0)),
            pl.BlockSpec((D_IN, D_H), lambda i: (0, 0)),
        ],
        out_specs=pl.BlockSpec((_RB, D_H), lambda i: (i, 0)),
        out_shape=jax.ShapeDtypeStruct((NP, D_H), jnp.float32),
    )(x, W)


def _tc_scale_split(xw, degraw):
    def body(xw_ref, dr_ref, y_ref):
        d = _deg_inv_sqrt(dr_ref)
        sc = xw_ref[...] * d
        y_ref[0] = sc[:, :128]
        y_ref[1] = sc[:, 128:]

    return pl.pallas_call(
        body,
        grid=(NP // _RB,),
        in_specs=[
            pl.BlockSpec((_RB, D_H), lambda i: (i, 0)),
            pl.BlockSpec((NC, _RB, L), lambda i: (0, i, 0)),
        ],
        out_specs=pl.BlockSpec((NC, _RB, 128), lambda i: (0, i, 0)),
        out_shape=jax.ShapeDtypeStruct((NC, NP, 128), jnp.float32),
    )(xw, degraw)


def _tc_layer2(acc1, y1, degraw, b1r, W2p):
    def body(a_ref, y_ref, dr_ref, b1_ref, w2_ref, y2_ref):
        d = _deg_inv_sqrt(dr_ref)
        h0 = jnp.maximum(d * (a_ref[0] + y_ref[0]) + b1_ref[0:1, :128], 0.0)
        h1 = jnp.maximum(d * (a_ref[1] + y_ref[1]) + b1_ref[0:1, 128:], 0.0)
        hw = lax.dot_general(
            h0, w2_ref[:128], (((1,), (0,)), ((), ())),
            precision=lax.Precision.HIGHEST, preferred_element_type=jnp.float32,
        ) + lax.dot_general(
            h1, w2_ref[128:], (((1,), (0,)), ((), ())),
            precision=lax.Precision.HIGHEST, preferred_element_type=jnp.float32,
        )
        y2_ref[...] = d * hw

    return pl.pallas_call(
        body,
        grid=(NP // _RB,),
        in_specs=[
            pl.BlockSpec((NC, _RB, 128), lambda i: (0, i, 0)),
            pl.BlockSpec((NC, _RB, 128), lambda i: (0, i, 0)),
            pl.BlockSpec((NC, _RB, L), lambda i: (0, i, 0)),
            pl.BlockSpec((1, D_H), lambda i: (0, 0)),
            pl.BlockSpec((D_H, L), lambda i: (0, 0)),
        ],
        out_specs=pl.BlockSpec((_RB, L), lambda i: (i, 0)),
        out_shape=jax.ShapeDtypeStruct((NP, L), jnp.float32),
    )(acc1, y1, degraw, b1r, W2p)


def _tc_out(acc2, y2, degraw, b2r):
    def body(a_ref, y_ref, dr_ref, b2_ref, o_ref):
        d = _deg_inv_sqrt(dr_ref)
        logit = d * (a_ref[0] + a_ref[1] + y_ref[...]) + b2_ref[...]
        lid = lax.broadcasted_iota(jnp.int32, (_RB, L), 1)
        real = lid < D_OUT
        masked = jnp.where(real, logit, -jnp.inf)
        m = jnp.max(masked, axis=1, keepdims=True)
        e = jnp.where(real, jnp.exp(logit - m), 0.0)
        p = e / jnp.sum(e, axis=1, keepdims=True)
        o_ref[...] = p[:, :D_OUT]

    return pl.pallas_call(
        body,
        grid=(NP // _RB,),
        in_specs=[
            pl.BlockSpec((NC, _RB, L), lambda i: (0, i, 0)),
            pl.BlockSpec((_RB, L), lambda i: (i, 0)),
            pl.BlockSpec((NC, _RB, L), lambda i: (0, i, 0)),
            pl.BlockSpec((1, L), lambda i: (0, 0)),
        ],
        out_specs=pl.BlockSpec((_RB, D_OUT), lambda i: (i, 0)),
        out_shape=jax.ShapeDtypeStruct((NP, D_OUT), jnp.float32),
    )(acc2, y2, degraw, b2r)


def kernel(x, edge_index, W1, b1, W2, b2):
    src = edge_index[0]
    dst = edge_index[1]
    x = jnp.pad(x, ((0, NP - N), (0, 0)))

    ones_blk = jnp.ones((CH, L), jnp.float32)
    zeros_narrow = jnp.zeros((ROWS_PER_TILE, L), jnp.float32)
    zeros_wide = jnp.zeros((ROWS_PER_TILE, 128), jnp.float32)
    b1r = b1.reshape(1, D_H)
    W2p = jnp.pad(W2, ((0, 0), (0, L - D_OUT)))
    b2r = jnp.pad(b2, (0, L - D_OUT)).reshape(1, L)

    degraw = _sc_deg(dst, ones_blk, zeros_narrow)
    xw = _tc_matmul(x, W1)
    y1 = _tc_scale_split(xw, degraw)
    acc1 = _sc_agg128(y1.reshape(NC * NP, 128), src, dst, zeros_wide)
    y2 = _tc_layer2(acc1, y1, degraw, b1r, W2p)
    acc2 = _sc_agg16(y2, src, dst, zeros_narrow)
    return _tc_out(acc2, y2, degraw, b2r)[:N]


# speedup vs baseline: 13.1954x; 1.1214x over previous
import functools

import jax
import jax.numpy as jnp
from jax import lax
from jax.experimental import pallas as pl
from jax.experimental.pallas import tpu as pltpu
from jax.experimental.pallas import tpu_sc as plsc

N = 10000
NP = 10240
E = 160000
D_IN = 256
D_H = 256
D_OUT = 4

NC, NS, L = 2, 16, 16
CH = 128
EP = 163840
NCHUNK = EP // CH
CPT_B = NCHUNK // NS
CPT_AC = NCHUNK // (NC * NS)
ROWS_PER_TILE = NP // NS

_mesh = plsc.VectorSubcoreMesh(
    core_axis_name="c", subcore_axis_name="s", num_cores=NC, num_subcores=NS
)
_sc_params = pltpu.CompilerParams(use_tc_tiling_on_sc=False)


def _sc_deg(dst2, ones_blk, zeros_narrow):
    @functools.partial(
        pl.kernel,
        out_type=jax.ShapeDtypeStruct((NC, NP, L), jnp.float32),
        mesh=_mesh,
        compiler_params=_sc_params,
        scratch_types=[
            pltpu.VMEM((CPT_AC, CH), jnp.int32),
            pltpu.VMEM((CH, L), jnp.float32),
            pltpu.VMEM_SHARED((NP, L), jnp.float32),
            pltpu.SemaphoreType.DMA,
        ],
    )
    def k(dst_hbm, ones_hbm, zero_hbm, out_hbm, idx_d2, ones_v, acc, sem):
        c = lax.axis_index("c")
        s = lax.axis_index("s")
        rows0 = s * ROWS_PER_TILE
        pltpu.sync_copy(zero_hbm, acc.at[pl.ds(rows0, ROWS_PER_TILE)])
        pltpu.sync_copy(ones_hbm, ones_v)
        ch0 = (c * NS + s) * CPT_AC
        pltpu.sync_copy(dst_hbm.at[pl.ds(ch0, CPT_AC)], idx_d2)
        plsc.subcore_barrier()

        FD = 8

        @pl.loop(0, CPT_AC, step=FD)
        def _(j):
            for b in range(FD):
                pltpu.async_copy(ones_v, acc.at[idx_d2.at[j + b]], sem, add=True)
            for b in range(FD):
                pltpu.make_async_copy(ones_v, acc.at[idx_d2.at[j + b]], sem).wait()

        plsc.subcore_barrier()
        pltpu.sync_copy(
            acc.at[pl.ds(rows0, ROWS_PER_TILE)],
            out_hbm.at[c].at[pl.ds(rows0, ROWS_PER_TILE)],
        )

    return k(dst2, ones_blk, zeros_narrow)


def _sc_agg(y_hbm_arr, src2, dst2, zeros_blk, width, dtype):
    @functools.partial(
        pl.kernel,
        out_type=jax.ShapeDtypeStruct((NC, NP, width), dtype),
        mesh=_mesh,
        compiler_params=_sc_params,
        scratch_types=[
            pltpu.VMEM((CPT_AC, CH), jnp.int32),
            pltpu.VMEM((CPT_AC, CH), jnp.int32),
            pltpu.VMEM((CH, width), dtype),
            pltpu.VMEM((CH, width), dtype),
            pltpu.VMEM_SHARED((NP, width), dtype),
            pltpu.SemaphoreType.DMA,
            pltpu.SemaphoreType.DMA,
        ],
    )
    def k(y_hbm, src_hbm, dst_hbm, zero_hbm, out_hbm,
          idx_s2, idx_d2, buf0, buf1, acc, gs0, gs1):
        c = lax.axis_index("c")
        s = lax.axis_index("s")
        rows0 = s * ROWS_PER_TILE
        pltpu.sync_copy(zero_hbm, acc.at[pl.ds(rows0, ROWS_PER_TILE)])
        ch0 = (c * NS + s) * CPT_AC
        pltpu.sync_copy(src_hbm.at[pl.ds(ch0, CPT_AC)], idx_s2)
        pltpu.sync_copy(dst_hbm.at[pl.ds(ch0, CPT_AC)], idx_d2)

        def g_start(j, buf, sem):
            pltpu.async_copy(y_hbm.at[idx_s2.at[j]], buf, sem)

        def g_wait(j, buf, sem):
            pltpu.make_async_copy(y_hbm.at[idx_s2.at[j]], buf, sem).wait()

        def s_add(j, buf):
            pltpu.sync_copy(buf, acc.at[idx_d2.at[j]], add=True)

        g_start(0, buf0, gs0)
        g_start(1, buf1, gs1)
        plsc.subcore_barrier()

        @pl.loop(0, CPT_AC, step=2)
        def _(j):
            g_wait(j, buf0, gs0)
            s_add(j, buf0)

            @pl.when(j + 2 < CPT_AC)
            def _():
                g_start(j + 2, buf0, gs0)

            g_wait(j + 1, buf1, gs1)
            s_add(j + 1, buf1)

            @pl.when(j + 3 < CPT_AC)
            def _():
                g_start(j + 3, buf1, gs1)

        plsc.subcore_barrier()
        pltpu.sync_copy(
            acc.at[pl.ds(rows0, ROWS_PER_TILE)],
            out_hbm.at[c].at[pl.ds(rows0, ROWS_PER_TILE)],
        )

    return k(y_hbm_arr, src2, dst2, zeros_blk)


_RB = 1024


def _deg_inv_sqrt(dr_ref):
    deg = dr_ref[0, :, 0:1] + dr_ref[1, :, 0:1] + 1.0
    return lax.rsqrt(deg)


def _tc_matmul(x, W):
    def body(x_ref, w_ref, o_ref):
        o_ref[...] = lax.dot_general(
            x_ref[...], w_ref[...], (((1,), (0,)), ((), ())),
            precision=lax.Precision.HIGHEST,
            preferred_element_type=jnp.float32,
        )

    return pl.pallas_call(
        body,
        grid=(NP // _RB,),
        in_specs=[
            pl.BlockSpec((_RB, D_IN), lambda i: (i, 0)),
            pl.BlockSpec((D_IN, D_H), lambda i: (0, 0)),
        ],
        out_specs=pl.BlockSpec((_RB, D_H), lambda i: (i, 0)),
        out_shape=jax.ShapeDtypeStruct((NP, D_H), jnp.float32),
    )(x, W)


def _tc_scale(xw, degraw):
    def body(xw_ref, dr_ref, y_ref):
        d = _deg_inv_sqrt(dr_ref)
        y_ref[...] = (xw_ref[...] * d).astype(jnp.bfloat16)

    return pl.pallas_call(
        body,
        grid=(NP // _RB,),
        in_specs=[
            pl.BlockSpec((_RB, D_H), lambda i: (i, 0)),
            pl.BlockSpec((NC, _RB, L), lambda i: (0, i, 0)),
        ],
        out_specs=pl.BlockSpec((_RB, D_H), lambda i: (i, 0)),
        out_shape=jax.ShapeDtypeStruct((NP, D_H), jnp.bfloat16),
    )(xw, degraw)


def _tc_layer2(acc1, y1, degraw, b1r, W2p):
    def body(a_ref, y_ref, dr_ref, b1_ref, w2_ref, y2_ref):
        d = _deg_inv_sqrt(dr_ref)
        t = (a_ref[0].astype(jnp.float32) + a_ref[1].astype(jnp.float32)
             + y_ref[...].astype(jnp.float32))
        h = jnp.maximum(d * t + b1_ref[...], 0.0)
        hw = lax.dot_general(
            h, w2_ref[...], (((1,), (0,)), ((), ())),
            precision=lax.Precision.HIGHEST, preferred_element_type=jnp.float32,
        )
        y2_ref[...] = d * hw

    return pl.pallas_call(
        body,
        grid=(NP // _RB,),
        in_specs=[
            pl.BlockSpec((NC, _RB, D_H), lambda i: (0, i, 0)),
            pl.BlockSpec((_RB, D_H), lambda i: (i, 0)),
            pl.BlockSpec((NC, _RB, L), lambda i: (0, i, 0)),
            pl.BlockSpec((1, D_H), lambda i: (0, 0)),
            pl.BlockSpec((D_H, L), lambda i: (0, 0)),
        ],
        out_specs=pl.BlockSpec((_RB, L), lambda i: (i, 0)),
        out_shape=jax.ShapeDtypeStruct((NP, L), jnp.float32),
    )(acc1, y1, degraw, b1r, W2p)


def _tc_out(acc2, y2, degraw, b2r):
    def body(a_ref, y_ref, dr_ref, b2_ref, o_ref):
        d = _deg_inv_sqrt(dr_ref)
        logit = d * (a_ref[0] + a_ref[1] + y_ref[...]) + b2_ref[...]
        lid = lax.broadcasted_iota(jnp.int32, (_RB, L), 1)
        real = lid < D_OUT
        masked = jnp.where(real, logit, -jnp.inf)
        m = jnp.max(masked, axis=1, keepdims=True)
        e = jnp.where(real, jnp.exp(logit - m), 0.0)
        p = e / jnp.sum(e, axis=1, keepdims=True)
        o_ref[...] = p[:, :D_OUT]

    return pl.pallas_call(
        body,
        grid=(NP // _RB,),
        in_specs=[
            pl.BlockSpec((NC, _RB, L), lambda i: (0, i, 0)),
            pl.BlockSpec((_RB, L), lambda i: (i, 0)),
            pl.BlockSpec((NC, _RB, L), lambda i: (0, i, 0)),
            pl.BlockSpec((1, L), lambda i: (0, 0)),
        ],
        out_specs=pl.BlockSpec((_RB, D_OUT), lambda i: (i, 0)),
        out_shape=jax.ShapeDtypeStruct((NP, D_OUT), jnp.float32),
    )(acc2, y2, degraw, b2r)


def kernel(x, edge_index, W1, b1, W2, b2):
    src = edge_index[0]
    dst = edge_index[1]
    pad = EP - E
    src2 = jnp.concatenate([src, jnp.zeros((pad,), jnp.int32)]).reshape(NCHUNK, CH)
    dst2 = jnp.concatenate([dst, jnp.full((pad,), N, jnp.int32)]).reshape(NCHUNK, CH)
    x = jnp.pad(x, ((0, NP - N), (0, 0)))

    ones_blk = jnp.ones((CH, L), jnp.float32)
    zeros_narrow = jnp.zeros((ROWS_PER_TILE, L), jnp.float32)
    zeros_wide = jnp.zeros((ROWS_PER_TILE, D_H), jnp.bfloat16)
    b1r = b1.reshape(1, D_H)
    W2p = jnp.pad(W2, ((0, 0), (0, L - D_OUT)))
    b2r = jnp.pad(b2, (0, L - D_OUT)).reshape(1, L)

    degraw = _sc_deg(dst2, ones_blk, zeros_narrow)
    xw = _tc_matmul(x, W1)
    y1 = _tc_scale(xw, degraw)
    acc1 = _sc_agg(y1, src2, dst2, zeros_wide, width=D_H, dtype=jnp.bfloat16)
    y2 = _tc_layer2(acc1, y1, degraw, b1r, W2p)
    acc2 = _sc_agg(y2, src2, dst2, zeros_narrow, width=L, dtype=jnp.float32)
    return _tc_out(acc2, y2, degraw, b2r)[:N]


# speedup vs baseline: 20.7786x; 1.5747x over previous
import functools

import jax
import jax.numpy as jnp
from jax import lax
from jax.experimental import pallas as pl
from jax.experimental.pallas import tpu as pltpu
from jax.experimental.pallas import tpu_sc as plsc

N = 10000
NP = 10240
E = 160000
D_IN = 256
D_H = 256
D_OUT = 4

NC, NS, L = 2, 16, 16
CH = 128
EP = 163840
NCHUNK = EP // CH
CPT_B = NCHUNK // NS
CPT_AC = NCHUNK // (NC * NS)
ROWS_PER_TILE = NP // NS

_mesh = plsc.VectorSubcoreMesh(
    core_axis_name="c", subcore_axis_name="s", num_cores=NC, num_subcores=NS
)
_sc_params = pltpu.CompilerParams(use_tc_tiling_on_sc=False)


def _sc_deg(dst2, ones_blk, zeros_narrow):
    @functools.partial(
        pl.kernel,
        out_type=jax.ShapeDtypeStruct((NC, NP, L), jnp.float32),
        mesh=_mesh,
        compiler_params=_sc_params,
        scratch_types=[
            pltpu.VMEM((CPT_AC, CH), jnp.int32),
            pltpu.VMEM((CH, L), jnp.float32),
            pltpu.VMEM_SHARED((NP, L), jnp.float32),
            pltpu.SemaphoreType.DMA,
        ],
    )
    def k(dst_hbm, ones_hbm, zero_hbm, out_hbm, idx_d2, ones_v, acc, sem):
        c = lax.axis_index("c")
        s = lax.axis_index("s")
        rows0 = s * ROWS_PER_TILE
        pltpu.sync_copy(zero_hbm, acc.at[pl.ds(rows0, ROWS_PER_TILE)])
        pltpu.sync_copy(ones_hbm, ones_v)
        ch0 = (c * NS + s) * CPT_AC
        pltpu.sync_copy(dst_hbm.at[pl.ds(ch0, CPT_AC)], idx_d2)
        plsc.subcore_barrier()

        FD = 8

        @pl.loop(0, CPT_AC, step=FD)
        def _(j):
            for b in range(FD):
                pltpu.async_copy(ones_v, acc.at[idx_d2.at[j + b]], sem, add=True)
            for b in range(FD):
                pltpu.make_async_copy(ones_v, acc.at[idx_d2.at[j + b]], sem).wait()

        plsc.subcore_barrier()
        pltpu.sync_copy(
            acc.at[pl.ds(rows0, ROWS_PER_TILE)],
            out_hbm.at[c].at[pl.ds(rows0, ROWS_PER_TILE)],
        )

    return k(dst2, ones_blk, zeros_narrow)


def _sc_agg(y_hbm_arr, src2, dst2, zeros_blk, width, dtype):
    @functools.partial(
        pl.kernel,
        out_type=jax.ShapeDtypeStruct((NC, NP, width), dtype),
        mesh=_mesh,
        compiler_params=_sc_params,
        scratch_types=[
            pltpu.VMEM((CPT_AC, CH), jnp.int32),
            pltpu.VMEM((CPT_AC, CH), jnp.int32),
            pltpu.VMEM((CH, width), dtype),
            pltpu.VMEM((CH, width), dtype),
            pltpu.VMEM_SHARED((NP, width), dtype),
            pltpu.SemaphoreType.DMA,
            pltpu.SemaphoreType.DMA,
        ],
    )
    def k(y_hbm, src_hbm, dst_hbm, zero_hbm, out_hbm,
          idx_s2, idx_d2, buf0, buf1, acc, gs0, gs1):
        c = lax.axis_index("c")
        s = lax.axis_index("s")
        rows0 = s * ROWS_PER_TILE
        pltpu.sync_copy(zero_hbm, acc.at[pl.ds(rows0, ROWS_PER_TILE)])
        ch0 = (c * NS + s) * CPT_AC
        pltpu.sync_copy(src_hbm.at[pl.ds(ch0, CPT_AC)], idx_s2)
        pltpu.sync_copy(dst_hbm.at[pl.ds(ch0, CPT_AC)], idx_d2)

        def g_start(j, buf, sem):
            pltpu.async_copy(y_hbm.at[idx_s2.at[j]], buf, sem)

        def g_wait(j, buf, sem):
            pltpu.make_async_copy(y_hbm.at[idx_s2.at[j]], buf, sem).wait()

        def s_add(j, buf):
            pltpu.sync_copy(buf, acc.at[idx_d2.at[j]], add=True)

        g_start(0, buf0, gs0)
        g_start(1, buf1, gs1)
        plsc.subcore_barrier()

        @pl.loop(0, CPT_AC, step=2)
        def _(j):
            g_wait(j, buf0, gs0)
            s_add(j, buf0)

            @pl.when(j + 2 < CPT_AC)
            def _():
                g_start(j + 2, buf0, gs0)

            g_wait(j + 1, buf1, gs1)
            s_add(j + 1, buf1)

            @pl.when(j + 3 < CPT_AC)
            def _():
                g_start(j + 3, buf1, gs1)

        plsc.subcore_barrier()
        pltpu.sync_copy(
            acc.at[pl.ds(rows0, ROWS_PER_TILE)],
            out_hbm.at[c].at[pl.ds(rows0, ROWS_PER_TILE)],
        )

    return k(y_hbm_arr, src2, dst2, zeros_blk)


_RB = 1024


def _deg_inv_sqrt(dr_ref):
    deg = dr_ref[0, :, 0:1] + dr_ref[1, :, 0:1] + 1.0
    return lax.rsqrt(deg)


def _tc_matmul(x, W):
    def body(x_ref, w_ref, o_ref):
        o_ref[...] = lax.dot_general(
            x_ref[...], w_ref[...], (((1,), (0,)), ((), ())),
            precision=lax.Precision.HIGHEST,
            preferred_element_type=jnp.float32,
        )

    return pl.pallas_call(
        body,
        grid=(NP // _RB,),
        in_specs=[
            pl.BlockSpec((_RB, D_IN), lambda i: (i, 0)),
            pl.BlockSpec((D_IN, D_H), lambda i: (0, 0)),
        ],
        out_specs=pl.BlockSpec((_RB, D_H), lambda i: (i, 0)),
        out_shape=jax.ShapeDtypeStruct((NP, D_H), jnp.float32),
    )(x, W)


def _tc_scale(xw, degraw):
    def body(xw_ref, dr_ref, y_ref):
        d = _deg_inv_sqrt(dr_ref)
        y_ref[...] = (xw_ref[...] * d).astype(jnp.bfloat16)

    return pl.pallas_call(
        body,
        grid=(NP // _RB,),
        in_specs=[
            pl.BlockSpec((_RB, D_H), lambda i: (i, 0)),
            pl.BlockSpec((NC, _RB, L), lambda i: (0, i, 0)),
        ],
        out_specs=pl.BlockSpec((_RB, D_H), lambda i: (i, 0)),
        out_shape=jax.ShapeDtypeStruct((NP, D_H), jnp.bfloat16),
    )(xw, degraw)


def _tc_layer2(acc1, y1, degraw, b1r, W2p):
    def body(a_ref, y_ref, dr_ref, b1_ref, w2_ref, y2_ref):
        d = _deg_inv_sqrt(dr_ref)
        t = (a_ref[0].astype(jnp.float32) + a_ref[1].astype(jnp.float32)
             + y_ref[...].astype(jnp.float32))
        h = jnp.maximum(d * t + b1_ref[...], 0.0)
        hw = lax.dot_general(
            h, w2_ref[...], (((1,), (0,)), ((), ())),
            precision=lax.Precision.HIGHEST, preferred_element_type=jnp.float32,
        )
        y2_ref[...] = d * hw

    return pl.pallas_call(
        body,
        grid=(NP // _RB,),
        in_specs=[
            pl.BlockSpec((NC, _RB, D_H), lambda i: (0, i, 0)),
            pl.BlockSpec((_RB, D_H), lambda i: (i, 0)),
            pl.BlockSpec((NC, _RB, L), lambda i: (0, i, 0)),
            pl.BlockSpec((1, D_H), lambda i: (0, 0)),
            pl.BlockSpec((D_H, L), lambda i: (0, 0)),
        ],
        out_specs=pl.BlockSpec((_RB, L), lambda i: (i, 0)),
        out_shape=jax.ShapeDtypeStruct((NP, L), jnp.float32),
    )(acc1, y1, degraw, b1r, W2p)


def _tc_out(acc2, y2, degraw, b2r):
    def body(a_ref, y_ref, dr_ref, b2_ref, o_ref):
        d = _deg_inv_sqrt(dr_ref)
        logit = d * (a_ref[0] + a_ref[1] + y_ref[...]) + b2_ref[...]
        lid = lax.broadcasted_iota(jnp.int32, (_RB, L), 1)
        real = lid < D_OUT
        masked = jnp.where(real, logit, -jnp.inf)
        m = jnp.max(masked, axis=1, keepdims=True)
        e = jnp.where(real, jnp.exp(logit - m), 0.0)
        p = e / jnp.sum(e, axis=1, keepdims=True)
        o_ref[...] = p[:, :D_OUT]

    return pl.pallas_call(
        body,
        grid=(NP // _RB,),
        in_specs=[
            pl.BlockSpec((NC, _RB, L), lambda i: (0, i, 0)),
            pl.BlockSpec((_RB, L), lambda i: (i, 0)),
            pl.BlockSpec((NC, _RB, L), lambda i: (0, i, 0)),
            pl.BlockSpec((1, L), lambda i: (0, 0)),
        ],
        out_specs=pl.BlockSpec((_RB, D_OUT), lambda i: (i, 0)),
        out_shape=jax.ShapeDtypeStruct((NP, D_OUT), jnp.float32),
    )(acc2, y2, degraw, b2r)


def kernel(x, edge_index, W1, b1, W2, b2):
    src = edge_index[0]
    dst = edge_index[1]
    pad = EP - E
    pad_i = jnp.arange(pad, dtype=jnp.int32)
    src2 = jnp.concatenate([src, pad_i % N]).reshape(NCHUNK, CH)
    dst2 = jnp.concatenate([dst, N + pad_i % (NP - N)]).reshape(NCHUNK, CH)
    x = jnp.pad(x, ((0, NP - N), (0, 0)))

    ones_blk = jnp.ones((CH, L), jnp.float32)
    zeros_narrow = jnp.zeros((ROWS_PER_TILE, L), jnp.float32)
    zeros_wide = jnp.zeros((ROWS_PER_TILE, D_H), jnp.bfloat16)
    b1r = b1.reshape(1, D_H)
    W2p = jnp.pad(W2, ((0, 0), (0, L - D_OUT)))
    b2r = jnp.pad(b2, (0, L - D_OUT)).reshape(1, L)

    degraw = _sc_deg(dst2, ones_blk, zeros_narrow)
    xw = _tc_matmul(x, W1)
    y1 = _tc_scale(xw, degraw)
    acc1 = _sc_agg(y1, src2, dst2, zeros_wide, width=D_H, dtype=jnp.bfloat16)
    y2 = _tc_layer2(acc1, y1, degraw, b1r, W2p)
    acc2 = _sc_agg(y2, src2, dst2, zeros_narrow, width=L, dtype=jnp.float32)
    return _tc_out(acc2, y2, degraw, b2r)[:N]
